# baseline (speedup 1.0000x reference)
import functools

import jax
import jax.numpy as jnp
from jax import lax
from jax.experimental import pallas as pl
from jax.experimental.pallas import tpu as pltpu
from jax.experimental.pallas import tpu_sc as plsc

N_EMB = 8192
DIM = 256
N_ROWS = 16384
COMMIT = 0.25

BM = 256
NBLK = N_ROWS // BM

SC_CORES = 2
SC_SUBCORES = 16
NW = SC_CORES * SC_SUBCORES
B_PER_W = N_ROWS // NW
CHUNK = 128
NCHUNK = B_PER_W // CHUNK


def _argmin_body(z_ref, e_ref, zsq_ref, esq_ref, idx_ref, loss_ref):
    i = pl.program_id(0)
    mm = lax.dot_general(
        z_ref[...], e_ref[...],
        dimension_numbers=(((1,), (1,)), ((), ())),
        preferred_element_type=jnp.float32,
    )
    red = esq_ref[...] - mm
    mind = jnp.min(red, axis=1, keepdims=True)
    jidx = lax.broadcasted_iota(jnp.int32, red.shape, 1)
    idx = jnp.min(jnp.where(red == mind, jidx, N_EMB), axis=1)
    idx_ref[0, 0, :] = idx

    @pl.when(i == 0)
    def _():
        loss_ref[0, 0] = 0.0

    loss_ref[0, 0] += 2.0 * jnp.sum(mind) + jnp.sum(zsq_ref[...])


def _tc_argmin(zb, eb, zsq, esq):
    return pl.pallas_call(
        _argmin_body,
        grid=(NBLK,),
        in_specs=[
            pl.BlockSpec((BM, DIM), lambda i: (i, 0)),
            pl.BlockSpec((N_EMB, DIM), lambda i: (0, 0)),
            pl.BlockSpec((1, 1, BM), lambda i: (i, 0, 0)),
            pl.BlockSpec((1, N_EMB), lambda i: (0, 0)),
        ],
        out_specs=[
            pl.BlockSpec((1, 1, BM), lambda i: (i, 0, 0)),
            pl.BlockSpec(memory_space=pltpu.SMEM),
        ],
        out_shape=[
            jax.ShapeDtypeStruct((NBLK, 1, BM), jnp.int32),
            jax.ShapeDtypeStruct((1, 1), jnp.float32),
        ],
    )(zb, eb, zsq, esq)


def _sc_gather(table, idx):
    mesh = plsc.VectorSubcoreMesh(core_axis_name="c", subcore_axis_name="s")

    @functools.partial(
        pl.kernel,
        mesh=mesh,
        out_type=jax.ShapeDtypeStruct((N_ROWS, DIM), jnp.float32),
        scratch_types=[
            pltpu.VMEM((NCHUNK, CHUNK), jnp.int32),
            pltpu.VMEM((2, CHUNK, DIM), jnp.float32),
            pltpu.SemaphoreType.DMA,
            pltpu.SemaphoreType.DMA,
        ],
    )
    def k(table_hbm, idx_hbm, out_hbm, idx_v, rows_v, gsem, osem):
        wid = lax.axis_index("s") * SC_CORES + lax.axis_index("c")
        base = wid * B_PER_W
        pltpu.sync_copy(idx_hbm.at[wid], idx_v)
        g0 = pltpu.async_copy(table_hbm.at[idx_v.at[0]], rows_v.at[0], gsem)
        g0.wait()
        for c in range(NCHUNK):
            buf = c % 2
            nxt = (c + 1) % 2
            if c + 1 < NCHUNK:
                gn = pltpu.async_copy(
                    table_hbm.at[idx_v.at[c + 1]], rows_v.at[nxt], gsem)
            ow = pltpu.async_copy(
                rows_v.at[buf], out_hbm.at[pl.ds(base + c * CHUNK, CHUNK)],
                osem)
            if c + 1 < NCHUNK:
                gn.wait()
            ow.wait()

    return k(table, idx)


def kernel(z, embeddings):
    zf = z.reshape(-1, DIM)
    zsq = jnp.sum(zf ** 2, axis=1).reshape(NBLK, 1, BM)
    esq = (0.5 * jnp.sum(embeddings ** 2, axis=1)).reshape(1, -1)
    zb = zf.astype(jnp.bfloat16)
    eb = embeddings.astype(jnp.bfloat16)
    idx, loss_sum = _tc_argmin(zb, eb, zsq, esq)
    z_q = _sc_gather(embeddings, idx.reshape(NW, NCHUNK, CHUNK))
    loss = (1.0 + COMMIT) * loss_sum[0, 0] / (N_ROWS * DIM)
    return (z_q, loss)


# speedup vs baseline: 1.0807x; 1.0807x over previous
import functools

import jax
import jax.numpy as jnp
from jax import lax
from jax.experimental import pallas as pl
from jax.experimental.pallas import tpu as pltpu
from jax.experimental.pallas import tpu_sc as plsc

N_EMB = 8192
DIM = 256
N_ROWS = 16384
COMMIT = 0.25

BM = 256
NBLK = N_ROWS // BM

SC_CORES = 2
SC_SUBCORES = 16
NW = SC_CORES * SC_SUBCORES
B_PER_W = N_ROWS // NW
CHUNK = 128
NCHUNK = B_PER_W // CHUNK


def _argmin_body(z_ref, e_ref, zsq_ref, esq_ref, idx_ref, loss_ref):
    i = pl.program_id(0)
    mm = lax.dot_general(
        z_ref[...], e_ref[...],
        dimension_numbers=(((1,), (1,)), ((), ())),
        preferred_element_type=jnp.float32,
    )
    red = esq_ref[...] - mm
    mind = jnp.min(red, axis=1, keepdims=True)
    jidx = lax.broadcasted_iota(jnp.int32, red.shape, 1).astype(jnp.float32)
    idxf = jnp.min(jnp.where(red == mind, jidx, float(N_EMB)), axis=1)
    idx_ref[0, 0, :] = idxf.astype(jnp.int32)

    @pl.when(i == 0)
    def _():
        loss_ref[0, 0] = 0.0

    loss_ref[0, 0] += 2.0 * jnp.sum(mind) + jnp.sum(zsq_ref[...])


def _tc_argmin(zb, eb, zsq, esq):
    return pl.pallas_call(
        _argmin_body,
        grid=(NBLK,),
        in_specs=[
            pl.BlockSpec((BM, DIM), lambda i: (i, 0)),
            pl.BlockSpec((N_EMB, DIM), lambda i: (0, 0)),
            pl.BlockSpec((1, 1, BM), lambda i: (i, 0, 0)),
            pl.BlockSpec((1, N_EMB), lambda i: (0, 0)),
        ],
        out_specs=[
            pl.BlockSpec((1, 1, BM), lambda i: (i, 0, 0)),
            pl.BlockSpec(memory_space=pltpu.SMEM),
        ],
        out_shape=[
            jax.ShapeDtypeStruct((NBLK, 1, BM), jnp.int32),
            jax.ShapeDtypeStruct((1, 1), jnp.float32),
        ],
    )(zb, eb, zsq, esq)


def _sc_gather(table, idx):
    mesh = plsc.VectorSubcoreMesh(core_axis_name="c", subcore_axis_name="s")

    @functools.partial(
        pl.kernel,
        mesh=mesh,
        out_type=jax.ShapeDtypeStruct((N_ROWS, DIM), jnp.float32),
        scratch_types=[
            pltpu.VMEM((NCHUNK, CHUNK), jnp.int32),
            pltpu.VMEM((2, CHUNK, DIM), jnp.float32),
            pltpu.SemaphoreType.DMA,
            pltpu.SemaphoreType.DMA,
        ],
    )
    def k(table_hbm, idx_hbm, out_hbm, idx_v, rows_v, gsem, osem):
        wid = lax.axis_index("s") * SC_CORES + lax.axis_index("c")
        base = wid * B_PER_W
        pltpu.sync_copy(idx_hbm.at[wid], idx_v)
        g0 = pltpu.async_copy(table_hbm.at[idx_v.at[0]], rows_v.at[0], gsem)
        g0.wait()
        for c in range(NCHUNK):
            buf = c % 2
            nxt = (c + 1) % 2
            if c + 1 < NCHUNK:
                gn = pltpu.async_copy(
                    table_hbm.at[idx_v.at[c + 1]], rows_v.at[nxt], gsem)
            ow = pltpu.async_copy(
                rows_v.at[buf], out_hbm.at[pl.ds(base + c * CHUNK, CHUNK)],
                osem)
            if c + 1 < NCHUNK:
                gn.wait()
            ow.wait()

    return k(table, idx)


def kernel(z, embeddings):
    zf = z.reshape(-1, DIM)
    zsq = jnp.sum(zf ** 2, axis=1).reshape(NBLK, 1, BM)
    esq = (0.5 * jnp.sum(embeddings ** 2, axis=1)).reshape(1, -1)
    zb = zf.astype(jnp.bfloat16)
    eb = embeddings.astype(jnp.bfloat16)
    idx, loss_sum = _tc_argmin(zb, eb, zsq, esq)
    z_q = _sc_gather(embeddings, idx.reshape(NW, NCHUNK, CHUNK))
    loss = (1.0 + COMMIT) * loss_sum[0, 0] / (N_ROWS * DIM)
    return (z_q, loss)
